# baseline (speedup 1.0000x reference)
import jax
import jax.numpy as jnp
from jax import lax
from jax.experimental import pallas as pl
from jax.experimental.pallas import tpu as pltpu
from jax.experimental.pallas import tpu_sc as plsc

NC = 2
NS = 16
NW = NC * NS
LANES = 16
CHUNK = 128


def _rsqrt16(v):
    i = plsc.bitcast(v, jnp.int32)
    i = jnp.int32(0x5F3759DF) - (i >> 1)
    y = plsc.bitcast(i, jnp.float32)
    half_v = v * 0.5
    for _ in range(3):
        y = y * (1.5 - half_v * y * y)
    return y


def _build(T, E, L, interpret=False):
    assert E == 128 and T % (NW * CHUNK) == 0
    LE = L * E
    n_chunks = T // (NW * CHUNK)
    assert n_chunks % 2 == 0
    per_w = n_chunks * CHUNK
    K = E // LANES

    mesh = plsc.VectorSubcoreMesh(
        core_axis_name="c", subcore_axis_name="s",
        num_cores=NC, num_subcores=NS)

    def body(seq_hbm, lab_hbm, table_hbm, seg_hbm, gamma_hbm, beta_hbm,
             pe_hbm, out_hbm,
             addend, rows0, rows1, xbuf0, xbuf1, idxall, laball, statsa, statsb,
             gbuf, bbuf, segbuf, semg0, semg1, semo0, semo1):
        wid = lax.axis_index("s") * NC + lax.axis_index("c")
        iota = lax.iota(jnp.int32, LANES)
        zeros = jnp.zeros((LANES,), jnp.float32)
        ones = jnp.ones((LANES,), jnp.float32)
        wbase = wid * per_w

        pltpu.sync_copy(seq_hbm.at[pl.ds(wbase, per_w)], idxall)
        pltpu.sync_copy(lab_hbm.at[pl.ds(wbase, per_w)], laball)
        pltpu.sync_copy(pe_hbm, addend.at[pl.ds(0, LE)])
        pltpu.sync_copy(pe_hbm, addend.at[pl.ds(LE, LE)])
        pltpu.sync_copy(seg_hbm, segbuf)
        pltpu.sync_copy(gamma_hbm, gbuf)
        pltpu.sync_copy(beta_hbm, bbuf)
        sg = [segbuf[pl.ds(E + k * LANES, LANES)] for k in range(K)]

        @plsc.parallel_loop(0, L, unroll=2)
        def _addseg(l):
            for k in range(K):
                off = LE + l * E + k * LANES
                addend[pl.ds(off, LANES)] = addend[pl.ds(off, LANES)] + sg[k]

        gvecs = [gbuf[pl.ds(k * LANES, LANES)] for k in range(K)]
        bvecs = [bbuf[pl.ds(k * LANES, LANES)] for k in range(K)]
        inv_e = jnp.float32(1.0 / E)

        def fire_gather(c, rows, sem):
            idx = idxall.at[pl.ds(c * CHUNK, CHUNK)]
            pltpu.async_copy(table_hbm.at[idx], rows, sem)

        def wait_gather(rows, sem):
            pltpu.make_async_copy(
                table_hbm.at[pl.ds(0, CHUNK)], rows, sem).wait()

        def fire_out(c, xbuf, sem):
            pltpu.async_copy(xbuf, out_hbm.at[pl.ds(c * CHUNK + wbase, CHUNK)],
                             sem)

        def wait_out(xbuf, sem):
            pltpu.make_async_copy(
                xbuf, out_hbm.at[pl.ds(wbase, CHUNK)], sem).wait()

        def compute_chunk(c, rows, xbuf):
            coff = c * CHUNK

            def group_body(g, gc):
                toff = coff + g * LANES
                seqv = idxall[pl.ds(toff, LANES)]
                labv = laball[pl.ds(toff, LANES)]
                maskf = jnp.where(seqv != 0, ones, zeros)
                tokv = g * LANES + iota
                lv = lax.rem(wbase + toff + iota, jnp.int32(L))
                abase = labv * LE + lv * E

                @plsc.parallel_loop(0, E, step=16, carry=(zeros, zeros))
                def sums(d, sc):
                    s, s2 = sc
                    xs = []
                    for i in range(16):
                        dd = (iota + (d + i)) & (E - 1)
                        tok = plsc.load_gather(rows, [tokv, dd])
                        ad = plsc.load_gather(addend, [abase + dd])
                        x = tok * maskf + ad
                        plsc.store_scatter(xbuf, [tokv, dd], x)
                        xs.append(x)

                    def tree(vs):
                        while len(vs) > 1:
                            vs = [a + b for a, b in zip(vs[::2], vs[1::2])]
                        return vs[0]
                    return (s + tree(xs), s2 + tree([x * x for x in xs]))
                s, s2 = sums
                mean = s * inv_e
                var = s2 * inv_e - mean * mean + jnp.float32(1e-12)
                rstd = _rsqrt16(var)
                statsa[pl.ds(g * LANES, LANES)] = rstd
                statsb[pl.ds(g * LANES, LANES)] = -mean * rstd
                return gc
            lax.fori_loop(0, CHUNK // LANES, group_body, 0)

            @plsc.parallel_loop(0, CHUNK, unroll=4)
            def _norm(t):
                tsp = jnp.full((LANES,), t, jnp.int32)
                av = plsc.load_gather(statsa, [tsp])
                bv = plsc.load_gather(statsb, [tsp])
                for k in range(K):
                    x = xbuf[t, pl.ds(k * LANES, LANES)]
                    y = (x * av + bv) * gvecs[k] + bvecs[k]
                    xbuf[t, pl.ds(k * LANES, LANES)] = y

        fire_gather(0, rows0, semg0)
        fire_out(0, xbuf0, semo0)
        fire_out(1, xbuf1, semo1)

        def pair_body(j, carry):
            c0 = 2 * j
            fire_gather(c0 + 1, rows1, semg1)
            wait_gather(rows0, semg0)
            wait_out(xbuf0, semo0)
            compute_chunk(c0, rows0, xbuf0)
            fire_out(c0, xbuf0, semo0)
            fire_gather(lax.rem(c0 + 2, n_chunks), rows0, semg0)
            wait_gather(rows1, semg1)
            wait_out(xbuf1, semo1)
            compute_chunk(c0 + 1, rows1, xbuf1)
            fire_out(c0 + 1, xbuf1, semo1)
            return carry
        lax.fori_loop(0, n_chunks // 2, pair_body, 0)
        wait_gather(rows0, semg0)
        wait_out(xbuf0, semo0)
        wait_out(xbuf1, semo1)

    return pl.kernel(
        body,
        out_type=jax.ShapeDtypeStruct((T, E), jnp.float32),
        mesh=mesh,
        scratch_types=[
            pltpu.VMEM((2 * LE,), jnp.float32),
            pltpu.VMEM((CHUNK, E), jnp.float32),
            pltpu.VMEM((CHUNK, E), jnp.float32),
            pltpu.VMEM((CHUNK, E), jnp.float32),
            pltpu.VMEM((CHUNK, E), jnp.float32),
            pltpu.VMEM((T // NW,), jnp.int32),
            pltpu.VMEM((T // NW,), jnp.int32),
            pltpu.VMEM((CHUNK,), jnp.float32),
            pltpu.VMEM((CHUNK,), jnp.float32),
            pltpu.VMEM((E,), jnp.float32),
            pltpu.VMEM((E,), jnp.float32),
            pltpu.VMEM((2 * E,), jnp.float32),
            pltpu.SemaphoreType.DMA,
            pltpu.SemaphoreType.DMA,
            pltpu.SemaphoreType.DMA,
            pltpu.SemaphoreType.DMA,
        ],
        compiler_params=pltpu.CompilerParams(needs_layout_passes=False),
        interpret=interpret,
    )


def kernel(sequence, segment_label, token_table, seg_table, gamma, beta, pe):
    B, L = sequence.shape
    V, E = token_table.shape
    seq = sequence.reshape(-1).astype(jnp.int32)
    lab = segment_label.reshape(-1).astype(jnp.int32)
    pe_l = pe[0, :L].reshape(-1)
    seg_flat = seg_table.reshape(-1)
    fn = _build(B * L, E, L)
    out = fn(seq, lab, token_table, seg_flat, gamma, beta, pe_l)
    return out.reshape(B, L, E)


# speedup vs baseline: 1.1347x; 1.1347x over previous
import jax
import jax.numpy as jnp
from jax import lax
from jax.experimental import pallas as pl
from jax.experimental.pallas import tpu as pltpu
from jax.experimental.pallas import tpu_sc as plsc

NC = 2
NS = 16
NW = NC * NS
LANES = 16
CHUNK = 128


def _rsqrt16(v):
    i = plsc.bitcast(v, jnp.int32)
    i = jnp.int32(0x5F3759DF) - (i >> 1)
    y = plsc.bitcast(i, jnp.float32)
    half_v = v * 0.5
    for _ in range(3):
        y = y * (1.5 - half_v * y * y)
    return y


def _build(T, E, L, interpret=False):
    assert E == 128 and T % (NW * CHUNK) == 0
    LE = L * E
    n_chunks = T // (NW * CHUNK)
    assert n_chunks % 2 == 0
    per_w = n_chunks * CHUNK
    K = E // LANES

    mesh = plsc.VectorSubcoreMesh(
        core_axis_name="c", subcore_axis_name="s",
        num_cores=NC, num_subcores=NS)

    def body(seq_hbm, lab_hbm, table_hbm, seg_hbm, gamma_hbm, beta_hbm,
             pe_hbm, out_hbm,
             addend, rows0, rows1, xbuf0, xbuf1, idxall, laball, statsa, statsb,
             gbuf, bbuf, segbuf, semg0, semg1, semo0, semo1):
        wid = lax.axis_index("s") * NC + lax.axis_index("c")
        iota = lax.iota(jnp.int32, LANES)
        zeros = jnp.zeros((LANES,), jnp.float32)
        ones = jnp.ones((LANES,), jnp.float32)
        wbase = wid * per_w

        pltpu.sync_copy(seq_hbm.at[pl.ds(wbase, per_w)], idxall)
        pltpu.sync_copy(lab_hbm.at[pl.ds(wbase, per_w)], laball)
        pltpu.sync_copy(pe_hbm, addend.at[pl.ds(0, LE)])
        pltpu.sync_copy(pe_hbm, addend.at[pl.ds(LE, LE)])
        pltpu.sync_copy(seg_hbm, segbuf)
        pltpu.sync_copy(gamma_hbm, gbuf)
        pltpu.sync_copy(beta_hbm, bbuf)
        sg = [segbuf[pl.ds(E + k * LANES, LANES)] for k in range(K)]

        @plsc.parallel_loop(0, L, unroll=2)
        def _addseg(l):
            for k in range(K):
                off = LE + l * E + k * LANES
                addend[pl.ds(off, LANES)] = addend[pl.ds(off, LANES)] + sg[k]

        gvecs = [gbuf[pl.ds(k * LANES, LANES)] for k in range(K)]
        bvecs = [bbuf[pl.ds(k * LANES, LANES)] for k in range(K)]
        inv_e = jnp.float32(1.0 / E)

        def fire_gather(c, rows, sem):
            idx = idxall.at[pl.ds(c * CHUNK, CHUNK)]
            pltpu.async_copy(table_hbm.at[idx], rows, sem)

        def wait_gather(rows, sem):
            pltpu.make_async_copy(
                table_hbm.at[pl.ds(0, CHUNK)], rows, sem).wait()

        def fire_out(c, xbuf, sem):
            pltpu.async_copy(xbuf, out_hbm.at[pl.ds(c * CHUNK + wbase, CHUNK)],
                             sem)

        def wait_out(xbuf, sem):
            pltpu.make_async_copy(
                xbuf, out_hbm.at[pl.ds(wbase, CHUNK)], sem).wait()

        def compute_chunk(c, rows, xbuf):
            coff = c * CHUNK

            def group_body(g, gc):
                toff = coff + g * LANES
                seqv = idxall[pl.ds(toff, LANES)]
                labv = laball[pl.ds(toff, LANES)]
                maskf = jnp.where(seqv != 0, ones, zeros)
                tokv = g * LANES + iota
                lv = lax.rem(wbase + toff + iota, jnp.int32(L))
                abase = labv * LE + lv * E

                @plsc.parallel_loop(0, E, unroll=16, carry=(zeros, zeros))
                def sums(d, sc):
                    s, s2 = sc
                    dd = iota ^ d
                    tok = plsc.load_gather(rows, [tokv, dd])
                    ad = plsc.load_gather(addend, [abase + dd])
                    x = tok * maskf + ad
                    plsc.store_scatter(xbuf, [tokv, dd], x)
                    return (s + x, s2 + x * x)
                s, s2 = sums
                mean = s * inv_e
                var = s2 * inv_e - mean * mean + jnp.float32(1e-12)
                rstd = _rsqrt16(var)
                statsa[pl.ds(g * LANES, LANES)] = rstd
                statsb[pl.ds(g * LANES, LANES)] = -mean * rstd
                return gc
            lax.fori_loop(0, CHUNK // LANES, group_body, 0)

            @plsc.parallel_loop(0, CHUNK, unroll=4)
            def _norm(t):
                tsp = jnp.full((LANES,), t, jnp.int32)
                av = plsc.load_gather(statsa, [tsp])
                bv = plsc.load_gather(statsb, [tsp])
                for k in range(K):
                    x = xbuf[t, pl.ds(k * LANES, LANES)]
                    y = (x * av + bv) * gvecs[k] + bvecs[k]
                    xbuf[t, pl.ds(k * LANES, LANES)] = y

        fire_gather(0, rows0, semg0)
        fire_out(0, xbuf0, semo0)
        fire_out(1, xbuf1, semo1)

        def pair_body(j, carry):
            c0 = 2 * j
            fire_gather(c0 + 1, rows1, semg1)
            wait_gather(rows0, semg0)
            wait_out(xbuf0, semo0)
            compute_chunk(c0, rows0, xbuf0)
            fire_out(c0, xbuf0, semo0)
            fire_gather(lax.rem(c0 + 2, n_chunks), rows0, semg0)
            wait_gather(rows1, semg1)
            wait_out(xbuf1, semo1)
            compute_chunk(c0 + 1, rows1, xbuf1)
            fire_out(c0 + 1, xbuf1, semo1)
            return carry
        lax.fori_loop(0, n_chunks // 2, pair_body, 0)
        wait_gather(rows0, semg0)
        wait_out(xbuf0, semo0)
        wait_out(xbuf1, semo1)

    return pl.kernel(
        body,
        out_type=jax.ShapeDtypeStruct((T, E), jnp.float32),
        mesh=mesh,
        scratch_types=[
            pltpu.VMEM((2 * LE,), jnp.float32),
            pltpu.VMEM((CHUNK, E), jnp.float32),
            pltpu.VMEM((CHUNK, E), jnp.float32),
            pltpu.VMEM((CHUNK, E), jnp.float32),
            pltpu.VMEM((CHUNK, E), jnp.float32),
            pltpu.VMEM((T // NW,), jnp.int32),
            pltpu.VMEM((T // NW,), jnp.int32),
            pltpu.VMEM((CHUNK,), jnp.float32),
            pltpu.VMEM((CHUNK,), jnp.float32),
            pltpu.VMEM((E,), jnp.float32),
            pltpu.VMEM((E,), jnp.float32),
            pltpu.VMEM((2 * E,), jnp.float32),
            pltpu.SemaphoreType.DMA,
            pltpu.SemaphoreType.DMA,
            pltpu.SemaphoreType.DMA,
            pltpu.SemaphoreType.DMA,
        ],
        compiler_params=pltpu.CompilerParams(needs_layout_passes=False),
        interpret=interpret,
    )


def kernel(sequence, segment_label, token_table, seg_table, gamma, beta, pe):
    B, L = sequence.shape
    V, E = token_table.shape
    seq = sequence.reshape(-1).astype(jnp.int32)
    lab = segment_label.reshape(-1).astype(jnp.int32)
    pe_l = pe[0, :L].reshape(-1)
    seg_flat = seg_table.reshape(-1)
    fn = _build(B * L, E, L)
    out = fn(seq, lab, token_table, seg_flat, gamma, beta, pe_l)
    return out.reshape(B, L, E)


# speedup vs baseline: 1.2229x; 1.0778x over previous
import jax
import jax.numpy as jnp
from jax import lax
from jax.experimental import pallas as pl
from jax.experimental.pallas import tpu as pltpu
from jax.experimental.pallas import tpu_sc as plsc

NC = 2
NS = 16
NW = NC * NS
LANES = 16
CHUNK = 80


def _rsqrt16(v):
    i = plsc.bitcast(v, jnp.int32)
    i = jnp.int32(0x5F3759DF) - (i >> 1)
    y = plsc.bitcast(i, jnp.float32)
    half_v = v * 0.5
    for _ in range(3):
        y = y * (1.5 - half_v * y * y)
    return y


def _build(T, E, L, interpret=False):
    assert E == 128 and T % (NW * CHUNK) == 0
    LE = L * E
    n_chunks = T // (NW * CHUNK)
    assert n_chunks % 2 == 0
    per_w = n_chunks * CHUNK
    K = E // LANES

    mesh = plsc.VectorSubcoreMesh(
        core_axis_name="c", subcore_axis_name="s",
        num_cores=NC, num_subcores=NS)

    def body(seq_hbm, lab_hbm, table_hbm, seg_hbm, gamma_hbm, beta_hbm,
             pe_hbm, out_hbm,
             addend, rows0, rows1, xbuf0, xbuf1, idxall, laball, statsa, statsb,
             gbuf, bbuf, segbuf, semg0, semg1, semo0, semo1):
        wid = lax.axis_index("s") * NC + lax.axis_index("c")
        iota = lax.iota(jnp.int32, LANES)
        zeros = jnp.zeros((LANES,), jnp.float32)
        ones = jnp.ones((LANES,), jnp.float32)
        wbase = wid * per_w

        pltpu.sync_copy(seq_hbm.at[pl.ds(wbase, per_w)], idxall)
        pltpu.sync_copy(lab_hbm.at[pl.ds(wbase, per_w)], laball)
        pltpu.sync_copy(pe_hbm, addend.at[pl.ds(0, LE)])
        pltpu.sync_copy(pe_hbm, addend.at[pl.ds(LE, LE)])
        pltpu.sync_copy(seg_hbm, segbuf)
        pltpu.sync_copy(gamma_hbm, gbuf)
        pltpu.sync_copy(beta_hbm, bbuf)
        sg = [segbuf[pl.ds(E + k * LANES, LANES)] for k in range(K)]
        for k in range(K):
            rows0[CHUNK, pl.ds(k * LANES, LANES)] = zeros
            rows1[CHUNK, pl.ds(k * LANES, LANES)] = zeros

        @plsc.parallel_loop(0, L, unroll=2)
        def _addseg(l):
            for k in range(K):
                off = LE + l * E + k * LANES
                addend[pl.ds(off, LANES)] = addend[pl.ds(off, LANES)] + sg[k]

        gvecs = [gbuf[pl.ds(k * LANES, LANES)] for k in range(K)]
        bvecs = [bbuf[pl.ds(k * LANES, LANES)] for k in range(K)]
        inv_e = jnp.float32(1.0 / E)
        dev = zeros
        for k in range(K):
            dev = dev + jnp.abs(gvecs[k] - ones) + jnp.abs(bvecs[k])
        gb_trivial = lax.reduce_max(dev, (0,)) == jnp.float32(0.0)

        def fire_gather(c, rows, sem):
            idx = idxall.at[pl.ds(c * CHUNK, CHUNK)]
            pltpu.async_copy(table_hbm.at[idx], rows.at[pl.ds(0, CHUNK)], sem)

        def wait_gather(rows, sem):
            pltpu.make_async_copy(
                table_hbm.at[pl.ds(0, CHUNK)], rows.at[pl.ds(0, CHUNK)],
                sem).wait()

        def fire_out(c, xbuf, sem):
            pltpu.async_copy(xbuf, out_hbm.at[pl.ds(c * CHUNK + wbase, CHUNK)],
                             sem)

        def wait_out(xbuf, sem):
            pltpu.make_async_copy(
                xbuf, out_hbm.at[pl.ds(wbase, CHUNK)], sem).wait()

        def compute_chunk(c, rows, xbuf):
            coff = c * CHUNK

            def group_body(g, gc):
                toff = coff + g * LANES
                seqv = idxall[pl.ds(toff, LANES)]
                labv = laball[pl.ds(toff, LANES)]
                tokv = g * LANES + iota
                tokg = jnp.where(seqv != 0, tokv, jnp.full((LANES,), CHUNK,
                                                           jnp.int32))
                lv = lax.rem(wbase + toff + iota, jnp.int32(L))
                abase = labv * LE + lv * E

                @plsc.parallel_loop(0, E, unroll=16, carry=(zeros, zeros))
                def sums(d, sc):
                    s, s2 = sc
                    dd = iota ^ d
                    tok = plsc.load_gather(rows, [tokg, dd])
                    ad = plsc.load_gather(addend, [abase + dd])
                    x = tok + ad
                    plsc.store_scatter(xbuf, [tokv, dd], x)
                    return (s + x, s2 + x * x)
                s, s2 = sums
                mean = s * inv_e
                var = s2 * inv_e - mean * mean + jnp.float32(1e-12)
                rstd = _rsqrt16(var)
                statsa[pl.ds(g * LANES, LANES)] = rstd
                statsb[pl.ds(g * LANES, LANES)] = -mean * rstd
                return gc
            lax.fori_loop(0, CHUNK // LANES, group_body, 0)

            def norm_pass(apply_gb):
                @plsc.parallel_loop(0, CHUNK, unroll=4)
                def _norm(t):
                    tsp = jnp.full((LANES,), t, jnp.int32)
                    av = plsc.load_gather(statsa, [tsp])
                    bv = plsc.load_gather(statsb, [tsp])
                    for k in range(K):
                        x = xbuf[t, pl.ds(k * LANES, LANES)]
                        y = x * av + bv
                        if apply_gb:
                            y = y * gvecs[k] + bvecs[k]
                        xbuf[t, pl.ds(k * LANES, LANES)] = y

            lax.cond(gb_trivial,
                     lambda: norm_pass(False),
                     lambda: norm_pass(True))

        fire_gather(0, rows0, semg0)
        fire_out(0, xbuf0, semo0)
        fire_out(1, xbuf1, semo1)

        def pair_body(j, carry):
            c0 = 2 * j
            fire_gather(c0 + 1, rows1, semg1)
            wait_gather(rows0, semg0)
            wait_out(xbuf0, semo0)
            compute_chunk(c0, rows0, xbuf0)
            fire_out(c0, xbuf0, semo0)
            fire_gather(lax.rem(c0 + 2, n_chunks), rows0, semg0)
            wait_gather(rows1, semg1)
            wait_out(xbuf1, semo1)
            compute_chunk(c0 + 1, rows1, xbuf1)
            fire_out(c0 + 1, xbuf1, semo1)
            return carry
        lax.fori_loop(0, n_chunks // 2, pair_body, 0)
        wait_gather(rows0, semg0)
        wait_out(xbuf0, semo0)
        wait_out(xbuf1, semo1)

    return pl.kernel(
        body,
        out_type=jax.ShapeDtypeStruct((T, E), jnp.float32),
        mesh=mesh,
        scratch_types=[
            pltpu.VMEM((2 * LE,), jnp.float32),
            pltpu.VMEM((CHUNK + 1, E), jnp.float32),
            pltpu.VMEM((CHUNK + 1, E), jnp.float32),
            pltpu.VMEM((CHUNK, E), jnp.float32),
            pltpu.VMEM((CHUNK, E), jnp.float32),
            pltpu.VMEM((T // NW,), jnp.int32),
            pltpu.VMEM((T // NW,), jnp.int32),
            pltpu.VMEM((CHUNK,), jnp.float32),
            pltpu.VMEM((CHUNK,), jnp.float32),
            pltpu.VMEM((E,), jnp.float32),
            pltpu.VMEM((E,), jnp.float32),
            pltpu.VMEM((2 * E,), jnp.float32),
            pltpu.SemaphoreType.DMA,
            pltpu.SemaphoreType.DMA,
            pltpu.SemaphoreType.DMA,
            pltpu.SemaphoreType.DMA,
        ],
        compiler_params=pltpu.CompilerParams(needs_layout_passes=False),
        interpret=interpret,
    )


def kernel(sequence, segment_label, token_table, seg_table, gamma, beta, pe):
    B, L = sequence.shape
    V, E = token_table.shape
    seq = sequence.reshape(-1).astype(jnp.int32)
    lab = segment_label.reshape(-1).astype(jnp.int32)
    pe_l = pe[0, :L].reshape(-1)
    seg_flat = seg_table.reshape(-1)
    fn = _build(B * L, E, L)
    out = fn(seq, lab, token_table, seg_flat, gamma, beta, pe_l)
    return out.reshape(B, L, E)
